# initial kernel scaffold (unmeasured)
import jax
import jax.numpy as jnp
from jax import lax
from jax.experimental import pallas as pl
from jax.experimental.pallas import tpu as pltpu

N_DEV = 4
B, SQ, SKV, HQ, DH = 2, 128, 512, 4, 64
D_MODEL = 512
D_QK = HQ * DH
CH = SKV // N_DEV
ROWS = B * CH


def kernel(x, Wq, K_ext, V_ext, Wo):
    def body(x_ref, wq_ref, k_ref, v_ref, wo_ref, out_ref,
             kv_comm, k_all, v_all, send_sems, recv_sems):
        my_pos = lax.axis_index("i")
        left = lax.rem(my_pos + N_DEV - 1, N_DEV)
        right = lax.rem(my_pos + 1, N_DEV)

        barrier_sem = pltpu.get_barrier_semaphore()
        for nbr in (left, right):
            pl.semaphore_signal(barrier_sem, inc=1, device_id=(nbr,),
                                device_id_type=pl.DeviceIdType.MESH)
        pl.semaphore_wait(barrier_sem, 2)

        k_loc = k_ref[...].astype(jnp.bfloat16).reshape(ROWS, D_QK)
        v_loc = v_ref[...].astype(jnp.bfloat16).reshape(ROWS, D_QK)
        kv_comm[0, :ROWS, :] = k_loc
        kv_comm[0, ROWS:, :] = v_loc
        k_all[my_pos, :, :] = k_loc
        v_all[my_pos, :, :] = v_loc

        for h in range(N_DEV - 1):
            s_slot, r_slot = h % 2, (h + 1) % 2
            rdma = pltpu.make_async_remote_copy(
                src_ref=kv_comm.at[s_slot],
                dst_ref=kv_comm.at[r_slot],
                send_sem=send_sems.at[s_slot],
                recv_sem=recv_sems.at[r_slot],
                device_id=(right,),
                device_id_type=pl.DeviceIdType.MESH,
            )
            rdma.start()
            rdma.wait()
            origin = lax.rem(my_pos - h - 1 + N_DEV, N_DEV)
            k_all[origin, :, :] = kv_comm[r_slot, :ROWS, :]
            v_all[origin, :, :] = kv_comm[r_slot, ROWS:, :]

        x_flat = x_ref[...].reshape(B * SQ, D_MODEL).astype(jnp.bfloat16)
        q_all = jnp.dot(x_flat, wq_ref[...].astype(jnp.bfloat16),
                        preferred_element_type=jnp.float32)

        qi = lax.broadcasted_iota(jnp.int32, (SQ, SKV), 0)
        ki = lax.broadcasted_iota(jnp.int32, (SQ, SKV), 1)
        qb, kb = qi // 64, ki // 64
        mask = (qb == kb) | (kb == 0) | ((qb + kb) % 3 == 0)

        wo = wo_ref[...].astype(jnp.bfloat16)
        for b in range(B):
            acc = jnp.zeros((SQ, D_MODEL), jnp.float32)
            for hh in range(HQ):
                q = q_all[b * SQ:(b + 1) * SQ,
                          hh * DH:(hh + 1) * DH].astype(jnp.bfloat16)
                kmat = jnp.concatenate(
                    [k_all[o, b * CH:(b + 1) * CH, hh * DH:(hh + 1) * DH]
                     for o in range(N_DEV)], axis=0)
                vmat = jnp.concatenate(
                    [v_all[o, b * CH:(b + 1) * CH, hh * DH:(hh + 1) * DH]
                     for o in range(N_DEV)], axis=0)
                s = lax.dot_general(
                    q, kmat, (((1,), (1,)), ((), ())),
                    preferred_element_type=jnp.float32) * 0.125
                s = jnp.where(mask, s, -1e9)
                m = jnp.max(s, axis=1, keepdims=True)
                w = jnp.exp(s - m)
                w = w / jnp.sum(w, axis=1, keepdims=True)
                ctx = jnp.dot(w.astype(jnp.bfloat16), vmat,
                              preferred_element_type=jnp.float32)
                acc = acc + jnp.dot(
                    ctx.astype(jnp.bfloat16), wo[hh * DH:(hh + 1) * DH, :],
                    preferred_element_type=jnp.float32)
            out_ref[b, :, :] = acc

    return pl.pallas_call(
        body,
        out_shape=jax.ShapeDtypeStruct((B, SQ, D_MODEL), jnp.float32),
        in_specs=[pl.BlockSpec(memory_space=pltpu.VMEM)] * 5,
        out_specs=pl.BlockSpec(memory_space=pltpu.VMEM),
        scratch_shapes=[
            pltpu.VMEM((2, 2 * ROWS, D_QK), jnp.bfloat16),
            pltpu.VMEM((N_DEV, ROWS, D_QK), jnp.bfloat16),
            pltpu.VMEM((N_DEV, ROWS, D_QK), jnp.bfloat16),
            pltpu.SemaphoreType.DMA((2,)),
            pltpu.SemaphoreType.DMA((2,)),
        ],
        compiler_params=pltpu.CompilerParams(collective_id=0),
    )(x, Wq, K_ext, V_ext, Wo)


# baseline (device time: 24619 ns/iter reference)
import jax
import jax.numpy as jnp
from jax import lax
from jax.experimental import pallas as pl
from jax.experimental.pallas import tpu as pltpu

N_DEV = 4
B, SQ, SKV, HQ, DH = 2, 128, 512, 4, 64
D_MODEL = 512
D_QK = HQ * DH
CH = SKV // N_DEV
ROWS = B * CH


def kernel(x, Wq, K_ext, V_ext, Wo):
    def body(x_ref, wq_ref, k_ref, v_ref, wo_ref, out_ref,
             kv_comm, k_all, v_all, send_sems, recv_sems):
        my_pos = lax.axis_index("i")
        left = lax.rem(my_pos + N_DEV - 1, N_DEV)
        right = lax.rem(my_pos + 1, N_DEV)

        barrier_sem = pltpu.get_barrier_semaphore()
        for nbr in (left, right):
            pl.semaphore_signal(barrier_sem, inc=1, device_id=(nbr,),
                                device_id_type=pl.DeviceIdType.MESH)
        pl.semaphore_wait(barrier_sem, 2)

        k_loc = k_ref[...].astype(jnp.bfloat16).reshape(ROWS, D_QK)
        v_loc = v_ref[...].astype(jnp.bfloat16).reshape(ROWS, D_QK)
        kv_comm[0, :ROWS, :] = k_loc
        kv_comm[0, ROWS:, :] = v_loc
        k_all[my_pos, :, :] = k_loc
        v_all[my_pos, :, :] = v_loc

        for h in range(N_DEV - 1):
            s_slot, r_slot = h % 2, (h + 1) % 2
            rdma = pltpu.make_async_remote_copy(
                src_ref=kv_comm.at[s_slot],
                dst_ref=kv_comm.at[r_slot],
                send_sem=send_sems.at[s_slot],
                recv_sem=recv_sems.at[r_slot],
                device_id=(right,),
                device_id_type=pl.DeviceIdType.MESH,
            )
            rdma.start()
            rdma.wait()
            origin = lax.rem(my_pos - h - 1 + N_DEV, N_DEV)
            k_all[origin, :, :] = kv_comm[r_slot, :ROWS, :]
            v_all[origin, :, :] = kv_comm[r_slot, ROWS:, :]

        x_flat = x_ref[...].reshape(B * SQ, D_MODEL)
        q_all = jnp.dot(x_flat, wq_ref[...],
                        preferred_element_type=jnp.float32)

        qi = lax.broadcasted_iota(jnp.int32, (SQ, SKV), 0)
        ki = lax.broadcasted_iota(jnp.int32, (SQ, SKV), 1)
        qb, kb = qi // 64, ki // 64
        mask = (qb == kb) | (kb == 0) | ((qb + kb) % 3 == 0)

        wo = wo_ref[...]
        for b in range(B):
            acc = jnp.zeros((SQ, D_MODEL), jnp.float32)
            for hh in range(HQ):
                q = q_all[b * SQ:(b + 1) * SQ, hh * DH:(hh + 1) * DH]
                kmat = jnp.concatenate(
                    [k_all[o, b * CH:(b + 1) * CH, hh * DH:(hh + 1) * DH]
                     for o in range(N_DEV)], axis=0).astype(jnp.float32)
                vmat = jnp.concatenate(
                    [v_all[o, b * CH:(b + 1) * CH, hh * DH:(hh + 1) * DH]
                     for o in range(N_DEV)], axis=0).astype(jnp.float32)
                s = lax.dot_general(
                    q, kmat, (((1,), (1,)), ((), ())),
                    preferred_element_type=jnp.float32) * 0.125
                s = jnp.where(mask, s, -1e9)
                m = jnp.max(s, axis=1, keepdims=True)
                w = jnp.exp(s - m)
                w = w / jnp.sum(w, axis=1, keepdims=True)
                ctx = jnp.dot(w, vmat,
                              preferred_element_type=jnp.float32)
                acc = acc + jnp.dot(
                    ctx, wo[hh * DH:(hh + 1) * DH, :],
                    preferred_element_type=jnp.float32)
            out_ref[b, :, :] = acc

    return pl.pallas_call(
        body,
        out_shape=jax.ShapeDtypeStruct((B, SQ, D_MODEL), jnp.float32),
        in_specs=[pl.BlockSpec(memory_space=pltpu.VMEM)] * 5,
        out_specs=pl.BlockSpec(memory_space=pltpu.VMEM),
        scratch_shapes=[
            pltpu.VMEM((2, 2 * ROWS, D_QK), jnp.bfloat16),
            pltpu.VMEM((N_DEV, ROWS, D_QK), jnp.bfloat16),
            pltpu.VMEM((N_DEV, ROWS, D_QK), jnp.bfloat16),
            pltpu.SemaphoreType.DMA((2,)),
            pltpu.SemaphoreType.DMA((2,)),
        ],
        compiler_params=pltpu.CompilerParams(collective_id=0),
    )(x, Wq, K_ext, V_ext, Wo)
